# baseline (speedup 1.0000x reference)
import functools

import jax
import jax.numpy as jnp
from jax import lax
from jax.experimental import pallas as pl
from jax.experimental.pallas import tpu as pltpu, tpu_sc as plsc

NUM_SETS = 32
SET_SIZE = 16
D = 128
NUM_GENES_USED = NUM_SETS * SET_SIZE
LANES = 16
DV = D // LANES


def _softmax_body(a_ref, o_ref):
    x = a_ref[...]
    m = jnp.max(x, axis=1, keepdims=True)
    e = jnp.exp(x - m)
    o_ref[...] = e / jnp.sum(e, axis=1, keepdims=True)


def _sc_body(gene_hbm, w_hbm, out_hbm, w_v, slab_v, out_v, sem):
    nc = 2
    wid = lax.axis_index("s") * nc + lax.axis_index("c")
    b = gene_hbm.shape[0]
    b_per_w = b // (nc * 16)

    pltpu.sync_copy(w_hbm, w_v)

    def batch_body(i, _):
        bb = wid * b_per_w + i
        cp = pltpu.make_async_copy(
            gene_hbm.at[bb, pl.ds(0, NUM_GENES_USED), :], slab_v, sem
        )
        cp.start()
        cp.wait()

        def set_body(s, _):
            wvec = w_v[s, :]
            accs = [jnp.zeros((LANES,), jnp.float32) for _ in range(DV)]
            for k in range(SET_SIZE):
                wk = wvec[k]
                row = s * SET_SIZE + k
                for v in range(DV):
                    accs[v] = accs[v] + wk * slab_v[row, pl.ds(v * LANES, LANES)]
            for v in range(DV):
                out_v[s, pl.ds(v * LANES, LANES)] = accs[v]
            return 0

        lax.fori_loop(0, NUM_SETS, set_body, 0)
        pltpu.sync_copy(out_v, out_hbm.at[bb])
        return 0

    lax.fori_loop(0, b_per_w, batch_body, 0)


def kernel(gene_features, attn_w):
    b = gene_features.shape[0]
    attn2 = attn_w.reshape(NUM_SETS, SET_SIZE)
    w = pl.pallas_call(
        _softmax_body,
        out_shape=jax.ShapeDtypeStruct((NUM_SETS, SET_SIZE), jnp.float32),
    )(attn2)

    mesh = plsc.VectorSubcoreMesh(core_axis_name="c", subcore_axis_name="s")
    f = pl.kernel(
        _sc_body,
        out_type=jax.ShapeDtypeStruct((b, NUM_SETS, D), jnp.float32),
        mesh=mesh,
        scratch_types=[
            pltpu.VMEM((NUM_SETS, SET_SIZE), jnp.float32),
            pltpu.VMEM((NUM_GENES_USED, D), jnp.float32),
            pltpu.VMEM((NUM_SETS, D), jnp.float32),
            pltpu.SemaphoreType.DMA,
        ],
    )
    return f(gene_features, w)


# speedup vs baseline: 1.1180x; 1.1180x over previous
import functools

import jax
import jax.numpy as jnp
from jax import lax
from jax.experimental import pallas as pl
from jax.experimental.pallas import tpu as pltpu, tpu_sc as plsc

NUM_SETS = 32
SET_SIZE = 16
D = 128
NUM_GENES_USED = NUM_SETS * SET_SIZE
LANES = 16
DV = D // LANES


def _softmax_body(a_ref, o_ref):
    x = a_ref[...]
    m = jnp.max(x, axis=1, keepdims=True)
    e = jnp.exp(x - m)
    o_ref[...] = e / jnp.sum(e, axis=1, keepdims=True)


HALF_SETS = NUM_SETS // 2
HALF_ROWS = NUM_GENES_USED // 2


def _sc_body(gene_hbm, w_hbm, out_hbm, w_v, slab_a, slab_b, out_v, sem_a, sem_b):
    nc = 2
    wid = lax.axis_index("s") * nc + lax.axis_index("c")
    b = gene_hbm.shape[0]
    b_per_w = b // (nc * 16)
    base = wid * b_per_w

    pltpu.sync_copy(w_hbm, w_v)

    def start_half(bb, h, buf, sem):
        pltpu.make_async_copy(
            gene_hbm.at[bb, pl.ds(h * HALF_ROWS, HALF_ROWS), :], buf, sem
        ).start()

    def compute_half(buf, h):
        def set_body(sl, _):
            wvec = w_v[h * HALF_SETS + sl, :]
            accs = [jnp.zeros((LANES,), jnp.float32) for _ in range(DV)]
            for k in range(SET_SIZE):
                wk = wvec[k]
                row = sl * SET_SIZE + k
                for v in range(DV):
                    accs[v] = accs[v] + wk * buf[row, pl.ds(v * LANES, LANES)]
            for v in range(DV):
                out_v[h * HALF_SETS + sl, pl.ds(v * LANES, LANES)] = accs[v]
            return 0

        lax.fori_loop(0, HALF_SETS, set_body, 0)

    start_half(base, 0, slab_a, sem_a)

    def batch_body(i, _):
        bb = base + i
        start_half(bb, 1, slab_b, sem_b)
        pltpu.make_async_copy(
            gene_hbm.at[bb, pl.ds(0, HALF_ROWS), :], slab_a, sem_a
        ).wait()
        compute_half(slab_a, 0)

        @pl.when(i < b_per_w - 1)
        def _():
            start_half(bb + 1, 0, slab_a, sem_a)

        pltpu.make_async_copy(
            gene_hbm.at[bb, pl.ds(HALF_ROWS, HALF_ROWS), :], slab_b, sem_b
        ).wait()
        compute_half(slab_b, 1)
        pltpu.sync_copy(out_v, out_hbm.at[bb])
        return 0

    lax.fori_loop(0, b_per_w, batch_body, 0)


def kernel(gene_features, attn_w):
    b = gene_features.shape[0]
    attn2 = attn_w.reshape(NUM_SETS, SET_SIZE)
    w = pl.pallas_call(
        _softmax_body,
        out_shape=jax.ShapeDtypeStruct((NUM_SETS, SET_SIZE), jnp.float32),
    )(attn2)

    mesh = plsc.VectorSubcoreMesh(core_axis_name="c", subcore_axis_name="s")
    f = pl.kernel(
        _sc_body,
        out_type=jax.ShapeDtypeStruct((b, NUM_SETS, D), jnp.float32),
        mesh=mesh,
        scratch_types=[
            pltpu.VMEM((NUM_SETS, SET_SIZE), jnp.float32),
            pltpu.VMEM((HALF_ROWS, D), jnp.float32),
            pltpu.VMEM((HALF_ROWS, D), jnp.float32),
            pltpu.VMEM((NUM_SETS, D), jnp.float32),
            pltpu.SemaphoreType.DMA,
            pltpu.SemaphoreType.DMA,
        ],
    )
    return f(gene_features, w)
